# baseline (speedup 1.0000x reference)
import functools

import jax
import jax.numpy as jnp
from jax import lax
from jax.experimental import pallas as pl
from jax.experimental.pallas import tpu as pltpu
from jax.experimental.pallas import tpu_sc as plsc

L = 16


def _tree(op, vals):
    vals = list(vals)
    while len(vals) > 1:
        nxt = [op(vals[i], vals[i + 1]) for i in range(0, len(vals) - 1, 2)]
        if len(vals) % 2:
            nxt.append(vals[-1])
        vals = nxt
    return vals[0]


@functools.lru_cache(maxsize=None)
def _build(B, IN, OUT, K, NOPS):
    mesh = plsc.VectorSubcoreMesh(core_axis_name="c", subcore_axis_name="s")
    NC, NS = mesh.num_cores, mesh.num_subcores
    NW = NC * NS
    assert B % NW == 0
    rows_per_w = B // NW
    R = 16 if rows_per_w % 16 == 0 else rows_per_w
    n_chunks = rows_per_w // R
    n_groups = OUT // L

    @functools.partial(
        pl.kernel,
        mesh=mesh,
        compiler_params=pltpu.CompilerParams(
            use_tc_tiling_on_sc=False, needs_layout_passes=False
        ),
        out_type=jax.ShapeDtypeStruct((B, OUT), jnp.float32),
        scratch_types=[
            pltpu.VMEM((K, OUT), jnp.int32),
            pltpu.VMEM((OUT * NOPS,), jnp.float32),
            pltpu.VMEM((OUT * K,), jnp.int32),
            pltpu.VMEM((OUT,), jnp.int32),
            pltpu.VMEM((OUT,), jnp.int32),
            pltpu.VMEM((OUT,), jnp.int32),
            pltpu.VMEM((R, IN), jnp.float32),
            pltpu.VMEM((R, OUT), jnp.float32),
        ],
    )
    def sc_kernel(x_hbm, wf_hbm, idxf_hbm, pos_hbm, out_hbm,
                  idx_v, wraw_v, iraw_v, id_v, ipos_v, perm_v, xbuf, obuf):
        wid = lax.axis_index("s") * NC + lax.axis_index("c")
        pltpu.sync_copy(wf_hbm, wraw_v)
        pltpu.sync_copy(idxf_hbm, iraw_v)
        pltpu.sync_copy(pos_hbm, ipos_v)

        one = jnp.full((L,), 1.0, jnp.float32)
        rows = [jnp.full((L,), r, jnp.int32) for r in range(R)]
        lanes = lax.iota(jnp.int32, L)

        def inv_body(g, _):
            s = pl.ds(g * L, L)
            plsc.store_scatter(perm_v, [ipos_v[s]], lanes + g * L)
            return _

        lax.fori_loop(0, n_groups, inv_body, 0, unroll=False)

        def opid_body(g, _):
            s = pl.ds(g * L, L)
            p = perm_v[s]
            wbase = p * NOPS
            w0 = plsc.load_gather(wraw_v, [wbase])
            w1 = plsc.load_gather(wraw_v, [wbase + 1])
            w2 = plsc.load_gather(wraw_v, [wbase + 2])
            w3 = plsc.load_gather(wraw_v, [wbase + 3])
            ibase = p * K
            for k in range(K):
                idx_v[k, s] = plsc.load_gather(iraw_v, [ibase + k])
            i0 = jnp.full((L,), 0, jnp.int32)
            b1 = w1 > w0
            m01 = jnp.maximum(w0, w1)
            i01 = jnp.where(b1, jnp.full((L,), 1, jnp.int32), i0)
            b2 = w2 > m01
            m012 = jnp.maximum(m01, w2)
            i012 = jnp.where(b2, jnp.full((L,), 2, jnp.int32), i01)
            b3 = w3 > m012
            id_v[s] = jnp.where(b3, jnp.full((L,), 3, jnp.int32), i012)
            return _

        lax.fori_loop(0, n_groups, opid_body, 0, unroll=False)

        for c in range(n_chunks):
            base = wid * rows_per_w + c * R
            pltpu.sync_copy(x_hbm.at[pl.ds(base, R)], xbuf)

            @plsc.parallel_loop(0, n_groups, 1)
            def group_body(g):
                s = pl.ds(g * L, L)
                idx = [idx_v[k, s] for k in range(K)]
                cols = perm_v[s]
                opid = id_v[s]
                sel_min = jnp.min(opid)
                sel_max = jnp.max(opid)
                sel = jnp.where(sel_min == sel_max, sel_min,
                                jnp.int32(NOPS))

                def uniform_arm(redop, post):
                    def arm():
                        for r in range(R):
                            gv = [plsc.load_gather(xbuf.at[r], [ik])
                                  for ik in idx]
                            plsc.store_scatter(
                                obuf, [rows[r], cols], post(_tree(redop, gv)))
                    return arm

                def coein_arm():
                    for r in range(R):
                        gv = [plsc.load_gather(xbuf.at[r], [ik])
                              for ik in idx]
                        q = _tree(lax.mul, [one - v for v in gv])
                        plsc.store_scatter(obuf, [rows[r], cols], one - q)

                def mixed_arm():
                    is_mx = opid == 1
                    is_co = opid == 3
                    is_pc = opid >= 2
                    for r in range(R):
                        gv = [plsc.load_gather(xbuf.at[r], [ik])
                              for ik in idx]
                        mn = _tree(jnp.minimum, gv)
                        mx = _tree(jnp.maximum, gv)
                        pr = _tree(lax.mul, gv)
                        q = _tree(lax.mul, [one - v for v in gv])
                        r01 = jnp.where(is_mx, mx, mn)
                        r23 = jnp.where(is_co, one - q, pr)
                        plsc.store_scatter(
                            obuf, [rows[r], cols], jnp.where(is_pc, r23, r01))

                lax.switch(sel, [
                    uniform_arm(jnp.minimum, lambda v: v),
                    uniform_arm(jnp.maximum, lambda v: v),
                    uniform_arm(lax.mul, lambda v: v),
                    coein_arm,
                    mixed_arm,
                ])
            pltpu.sync_copy(obuf, out_hbm.at[pl.ds(base, R)])

    return sc_kernel


def kernel(x, weights, connection_indices):
    B, IN = x.shape
    OUT, NOPS = weights.shape
    K = connection_indices.shape[1]
    sc_kernel = _build(B, IN, OUT, K, NOPS)
    opid = jnp.argmax(weights, axis=-1)
    oh = (opid[:, None] == jnp.arange(NOPS, dtype=opid.dtype)[None, :])
    ohi = oh.astype(jnp.int32)
    counts = jnp.sum(ohi, axis=0)
    offsets = jnp.concatenate(
        [jnp.zeros((1,), jnp.int32), jnp.cumsum(counts)[:-1]])
    pos_all = jnp.cumsum(ohi, axis=0) - ohi + offsets[None, :]
    pos = jnp.sum(jnp.where(oh, pos_all, 0), axis=1)
    return sc_kernel(x, weights.reshape(-1), connection_indices.reshape(-1),
                     pos)
